# baseline (speedup 1.0000x reference)
import functools

import jax
import jax.numpy as jnp
from jax.experimental import pallas as pl
from jax.experimental.pallas import tpu as pltpu

_ALPHA = 0.2
_NOISE_STD = 0.05


def _round_up(x, m):
    return ((x + m - 1) // m) * m


def _rnn_kernel(x_ref, noise_ref, wi_ref, wrec_ref, wo_ref, h0_ref, out_ref,
                h_c, r_c, buf_ref, *, bp, ts):
    H = wrec_ref.shape[0]
    hb = bp // 2

    @pl.when(pl.program_id(0) == 0)
    def _init():
        h0b = jnp.broadcast_to(h0_ref[...], (bp, H))
        h_c[...] = h0b
        r_c[...] = jnp.tanh(h0b).astype(jnp.bfloat16)

    inp = jnp.dot(x_ref[...], wi_ref[...], preferred_element_type=jnp.float32)
    n2d = noise_ref[...].reshape(ts * bp, H)
    buf_ref[...] = (_NOISE_STD * n2d + inp).astype(jnp.bfloat16)

    wrec = wrec_ref[...]

    def step(j, carry):
        h0, h1, r0, r1 = carry
        off = pl.multiple_of(j * bp, bp)
        rec0 = jnp.dot(r0, wrec, preferred_element_type=jnp.float32)
        rec1 = jnp.dot(r1, wrec, preferred_element_type=jnp.float32)
        d0 = buf_ref[pl.ds(off, hb), :].astype(jnp.float32)
        d1 = buf_ref[pl.ds(off + hb, hb), :].astype(jnp.float32)
        h0n = (1.0 - _ALPHA) * h0 + d0 + rec0
        h1n = (1.0 - _ALPHA) * h1 + d1 + rec1
        r0n = jnp.tanh(h0n).astype(jnp.bfloat16)
        r1n = jnp.tanh(h1n).astype(jnp.bfloat16)
        buf_ref[pl.ds(off, hb), :] = r0n
        buf_ref[pl.ds(off + hb, hb), :] = r1n
        return (h0n, h1n, r0n, r1n)

    init = (h_c[pl.ds(0, hb), :], h_c[pl.ds(hb, hb), :],
            r_c[pl.ds(0, hb), :], r_c[pl.ds(hb, hb), :])
    h0f, h1f, r0f, r1f = jax.lax.fori_loop(0, ts, step, init, unroll=8)
    h_c[pl.ds(0, hb), :] = h0f
    h_c[pl.ds(hb, hb), :] = h1f
    r_c[pl.ds(0, hb), :] = r0f
    r_c[pl.ds(hb, hb), :] = r1f

    out = jnp.dot(buf_ref[...], wo_ref[...],
                  preferred_element_type=jnp.float32)
    out_ref[...] = out.reshape(ts, bp, out.shape[-1])


def kernel(x, noise_tm, wi, si, wrec, wo, so, h0):
    B, S, I = x.shape
    H = wrec.shape[0]
    O = wo.shape[1]

    wi_a = (_ALPHA * (wi * si[:, None])).astype(jnp.bfloat16)
    wrec_ta = (_ALPHA * jnp.transpose(wrec)).astype(jnp.bfloat16)
    wo_full = (wo * so[None, :]).astype(jnp.bfloat16)
    h0_2d = h0.reshape(1, H).astype(jnp.float32)

    Bp = _round_up(max(B, 8), 8)

    TS = min(64, S)
    S_pad = _round_up(S, TS)
    NT = S_pad // TS

    x_p = jnp.pad(x, ((0, Bp - B), (0, S_pad - S), (0, 0)))
    x2d = jnp.transpose(x_p, (1, 0, 2)).reshape(S_pad * Bp, I)
    x2d = x2d.astype(jnp.bfloat16)

    n_p = jnp.pad(noise_tm, ((0, S_pad - S), (0, Bp - B), (0, 0)))

    _kernel_fn = functools.partial(_rnn_kernel, bp=Bp, ts=TS)

    grid_spec = pltpu.PrefetchScalarGridSpec(
        num_scalar_prefetch=0,
        grid=(NT,),
        in_specs=[
            pl.BlockSpec((TS * Bp, I), lambda t: (t, 0)),
            pl.BlockSpec((TS, Bp, H), lambda t: (t, 0, 0)),
            pl.BlockSpec((I, H), lambda t: (0, 0)),
            pl.BlockSpec((H, H), lambda t: (0, 0)),
            pl.BlockSpec((H, O), lambda t: (0, 0)),
            pl.BlockSpec((1, H), lambda t: (0, 0)),
        ],
        out_specs=pl.BlockSpec((TS, Bp, O), lambda t: (t, 0, 0)),
        scratch_shapes=[
            pltpu.VMEM((Bp, H), jnp.float32),
            pltpu.VMEM((Bp, H), jnp.bfloat16),
            pltpu.VMEM((TS * Bp, H), jnp.bfloat16),
        ],
    )

    out_tm = pl.pallas_call(
        _kernel_fn,
        out_shape=jax.ShapeDtypeStruct((S_pad, Bp, O), jnp.float32),
        grid_spec=grid_spec,
        compiler_params=pltpu.CompilerParams(
            dimension_semantics=("arbitrary",),
            vmem_limit_bytes=40 * 2**20),
    )(x2d, n_p, wi_a, wrec_ta, wo_full, h0_2d)

    out = out_tm[:S, :B]
    return jnp.transpose(out, (1, 0, 2))


# speedup vs baseline: 1.5134x; 1.5134x over previous
import functools

import jax
import jax.numpy as jnp
from jax.experimental import pallas as pl
from jax.experimental.pallas import tpu as pltpu

_ALPHA = 0.2
_NOISE_STD = 0.05


def _round_up(x, m):
    return ((x + m - 1) // m) * m


def _rnn_kernel(x_hbm, noise_ref, wi_ref, wrec_ref, wo_ref, h0_ref, out_hbm,
                h_c, r_c, buf_ref, x_buf, o_buf, x_sem, o_sem, *, b, bp, ts,
                nt):
    H = wrec_ref.shape[0]
    t = pl.program_id(0)
    slot = jax.lax.rem(t, 2)

    def x_tile_dma(sl, tile, rowb):
        return pltpu.make_async_copy(
            x_hbm.at[rowb, pl.ds(tile * ts, ts), :],
            x_buf.at[sl, :, rowb, :],
            x_sem.at[sl])

    def o_tile_dma(sl, tile, rowb):
        return pltpu.make_async_copy(
            o_buf.at[sl, :, rowb, :],
            out_hbm.at[rowb, pl.ds(tile * ts, ts), :],
            o_sem.at[sl])

    @pl.when(t == 0)
    def _prologue():
        h0b = jnp.broadcast_to(h0_ref[...], (bp, H))
        h_c[...] = h0b
        r_c[...] = jnp.tanh(h0b)
        for rb in range(b):
            x_tile_dma(0, 0, rb).start()

    @pl.when(t + 1 < nt)
    def _prefetch_next_x():
        for rb in range(b):
            x_tile_dma((t + 1) % 2, t + 1, rb).start()

    for rb in range(b):
        x_tile_dma(slot, 0, rb).wait()
    x2d = x_buf[slot].reshape(ts * bp, x_buf.shape[-1])
    inp = jnp.dot(x2d, wi_ref[...], preferred_element_type=jnp.float32)
    n2d = noise_ref[...].reshape(ts * bp, H)
    buf_ref[...] = _NOISE_STD * n2d + inp

    wrec = wrec_ref[...]

    def step(j, carry):
        h, r = carry
        off = pl.multiple_of(j * bp, bp)
        rec = jnp.dot(r, wrec, preferred_element_type=jnp.float32)
        h_new = (1.0 - _ALPHA) * h + buf_ref[pl.ds(off, bp), :] + rec
        r_new = jnp.tanh(h_new)
        buf_ref[pl.ds(off, bp), :] = r_new
        return (h_new, r_new)

    h_fin, r_fin = jax.lax.fori_loop(0, ts, step, (h_c[...], r_c[...]),
                                     unroll=8)
    h_c[...] = h_fin
    r_c[...] = r_fin

    @pl.when(t >= 2)
    def _drain_old_out():
        for rb in range(b):
            o_tile_dma(slot, 0, rb).wait()

    out = jnp.dot(buf_ref[...], wo_ref[...],
                  preferred_element_type=jnp.float32)
    o_buf[slot] = out.reshape(ts, bp, out.shape[-1])
    for rb in range(b):
        o_tile_dma(slot, t, rb).start()

    @pl.when(t == nt - 1)
    def _epilogue():
        for rb in range(b):
            o_tile_dma(slot, 0, rb).wait()
        if nt > 1:
            for rb in range(b):
                o_tile_dma((nt - 2) % 2, 0, rb).wait()


def kernel(x, noise_tm, wi, si, wrec, wo, so, h0):
    B, S, I = x.shape
    H = wrec.shape[0]
    O = wo.shape[1]

    wi_a = (_ALPHA * (wi * si[:, None])).astype(jnp.float32)
    wrec_ta = (_ALPHA * jnp.transpose(wrec)).astype(jnp.float32)
    wo_full = (wo * so[None, :]).astype(jnp.float32)
    h0_2d = h0.reshape(1, H).astype(jnp.float32)

    Bp = _round_up(max(B, 8), 8)

    TS = min(64, S)
    S_pad = _round_up(S, TS)
    NT = S_pad // TS

    x_p = jnp.pad(x, ((0, 0), (0, S_pad - S), (0, 0)))
    n_p = jnp.pad(noise_tm, ((0, S_pad - S), (0, Bp - B), (0, 0)))

    _kernel_fn = functools.partial(_rnn_kernel, b=B, bp=Bp, ts=TS, nt=NT)

    grid_spec = pltpu.PrefetchScalarGridSpec(
        num_scalar_prefetch=0,
        grid=(NT,),
        in_specs=[
            pl.BlockSpec(memory_space=pltpu.MemorySpace.HBM),
            pl.BlockSpec((TS, Bp, H), lambda t: (t, 0, 0)),
            pl.BlockSpec((I, H), lambda t: (0, 0)),
            pl.BlockSpec((H, H), lambda t: (0, 0)),
            pl.BlockSpec((H, O), lambda t: (0, 0)),
            pl.BlockSpec((1, H), lambda t: (0, 0)),
        ],
        out_specs=pl.BlockSpec(memory_space=pltpu.MemorySpace.HBM),
        scratch_shapes=[
            pltpu.VMEM((Bp, H), jnp.float32),
            pltpu.VMEM((Bp, H), jnp.float32),
            pltpu.VMEM((TS * Bp, H), jnp.float32),
            pltpu.VMEM((2, TS, Bp, I), jnp.float32),
            pltpu.VMEM((2, TS, Bp, O), jnp.float32),
            pltpu.SemaphoreType.DMA((2,)),
            pltpu.SemaphoreType.DMA((2,)),
        ],
    )

    out_p = pl.pallas_call(
        _kernel_fn,
        out_shape=jax.ShapeDtypeStruct((B, S_pad, O), jnp.float32),
        grid_spec=grid_spec,
        compiler_params=pltpu.CompilerParams(
            dimension_semantics=("arbitrary",),
            vmem_limit_bytes=56 * 2**20),
    )(x_p, n_p, wi_a, wrec_ta, wo_full, h0_2d)

    return out_p[:, :S]


# speedup vs baseline: 1.5491x; 1.0236x over previous
import functools

import jax
import jax.numpy as jnp
from jax.experimental import pallas as pl
from jax.experimental.pallas import tpu as pltpu

_ALPHA = 0.2
_NOISE_STD = 0.05


def _round_up(x, m):
    return ((x + m - 1) // m) * m


def _rnn_kernel(x_hbm, noise_ref, wi_ref, wrec_ref, wo_ref, h0_ref, out_hbm,
                h_c, r_c, buf_ref, rate_ref, x_buf, o_buf, x_sem, o_sem,
                *, b, bp, ts, nt):
    H = wrec_ref.shape[0]
    I = wi_ref.shape[0]
    O = o_buf.shape[-1]
    t = pl.program_id(0)
    slot = jax.lax.rem(t, 2)
    hb = bp // 2
    f32 = jnp.float32

    def x_tile_dma(sl, tile, rowb):
        return pltpu.make_async_copy(
            x_hbm.at[rowb, pl.ds(tile * ts, ts), :],
            x_buf.at[sl, :, rowb, :],
            x_sem.at[sl])

    def o_tile_dma(sl, tile, rowb):
        return pltpu.make_async_copy(
            o_buf.at[sl, :, rowb, :],
            out_hbm.at[rowb, pl.ds(tile * ts, ts), :],
            o_sem.at[sl])

    @pl.when(t == 0)
    def _prologue():
        h0b = jnp.broadcast_to(h0_ref[...], (bp, H))
        h_c[...] = h0b
        r_c[...] = jnp.tanh(h0b).astype(jnp.bfloat16)
        for rb in range(b):
            x_tile_dma(0, 0, rb).start()

    @pl.when(t + 1 < nt)
    def _prefetch_next_x():
        for rb in range(b):
            x_tile_dma((t + 1) % 2, t + 1, rb).start()

    for rb in range(b):
        x_tile_dma(slot, 0, rb).wait()

    pltpu.matmul_push_rhs(wi_ref[:, 0:256], staging_register=0, mxu_index=0)
    pltpu.matmul_push_rhs(wi_ref[:, 256:512], staging_register=0, mxu_index=1)

    n_chunks = (ts * bp) // 256
    tpc = 256 // bp

    def _drive_chunk(m):
        addr = (m % 4) * 64
        p0 = pltpu.matmul_pop(addr, (256, 256), f32, 0)
        p1 = pltpu.matmul_pop(addr, (256, 256), f32, 1)
        inp = jnp.concatenate([p0, p1], axis=1)
        nch = noise_ref[pl.ds(m * tpc, tpc), :, :].reshape(256, H)
        buf_ref[pl.ds(m * 256, 256), :] = _NOISE_STD * nch + inp

    for m in range(n_chunks):
        addr = (m % 4) * 64
        lhs = x_buf[slot, pl.ds(m * tpc, tpc), :, :].reshape(256, I)
        pltpu.matmul_acc_lhs(addr, lhs, 0,
                             load_staged_rhs=(0 if m == 0 else None))
        pltpu.matmul_acc_lhs(addr, lhs, 1,
                             load_staged_rhs=(0 if m == 0 else None))
        if m > 0:
            _drive_chunk(m - 1)
    _drive_chunk(n_chunks - 1)

    def push_wrec():
        pltpu.matmul_push_rhs(wrec_ref[0:256, 0:256], staging_register=0,
                              mxu_index=0)
        pltpu.matmul_push_rhs(wrec_ref[256:512, 0:256], staging_register=1,
                              mxu_index=0)
        pltpu.matmul_push_rhs(wrec_ref[0:256, 256:512], staging_register=0,
                              mxu_index=1)
        pltpu.matmul_push_rhs(wrec_ref[256:512, 256:512], staging_register=1,
                              mxu_index=1)

    def rec_step(j, carry, push_next):
        h0, h1, r0, r1 = carry
        off = pl.multiple_of(j * bp, bp)
        pltpu.matmul_acc_lhs(0, r0[:, 0:256], 0, load_staged_rhs=0)
        pltpu.matmul_acc_lhs(8, r1[:, 0:256], 0, load_staged_rhs=None)
        pltpu.matmul_acc_lhs(0, r0[:, 256:512], 0, load_staged_rhs=1)
        pltpu.matmul_acc_lhs(8, r1[:, 256:512], 0, load_staged_rhs=None)
        pltpu.matmul_acc_lhs(0, r0[:, 0:256], 1, load_staged_rhs=0)
        pltpu.matmul_acc_lhs(8, r1[:, 0:256], 1, load_staged_rhs=None)
        pltpu.matmul_acc_lhs(0, r0[:, 256:512], 1, load_staged_rhs=1)
        pltpu.matmul_acc_lhs(8, r1[:, 256:512], 1, load_staged_rhs=None)
        if push_next:
            push_wrec()
        rec0 = jnp.concatenate([pltpu.matmul_pop(0, (hb, 256), f32, 0),
                                pltpu.matmul_pop(0, (hb, 256), f32, 1)],
                               axis=1)
        h0n = (1.0 - _ALPHA) * h0 + buf_ref[pl.ds(off, hb), :] + rec0
        r0n = jnp.tanh(h0n).astype(jnp.bfloat16)
        rate_ref[pl.ds(off, hb), :] = r0n
        rec1 = jnp.concatenate([pltpu.matmul_pop(8, (hb, 256), f32, 0),
                                pltpu.matmul_pop(8, (hb, 256), f32, 1)],
                               axis=1)
        h1n = (1.0 - _ALPHA) * h1 + buf_ref[pl.ds(off + hb, hb), :] + rec1
        r1n = jnp.tanh(h1n).astype(jnp.bfloat16)
        rate_ref[pl.ds(off + hb, hb), :] = r1n
        return (h0n, h1n, r0n, r1n)

    push_wrec()
    init = (h_c[pl.ds(0, hb), :], h_c[pl.ds(hb, hb), :],
            r_c[pl.ds(0, hb), :], r_c[pl.ds(hb, hb), :])
    carry = jax.lax.fori_loop(
        0, ts - 1, lambda j, c: rec_step(j, c, True), init, unroll=8)
    h0f, h1f, r0f, r1f = rec_step(ts - 1, carry, False)
    h_c[pl.ds(0, hb), :] = h0f
    h_c[pl.ds(hb, hb), :] = h1f
    r_c[pl.ds(0, hb), :] = r0f
    r_c[pl.ds(hb, hb), :] = r1f

    @pl.when(t >= 2)
    def _drain_old_out():
        for rb in range(b):
            o_tile_dma(slot, 0, rb).wait()

    o_chunks = (ts * bp) // 512
    otpc = 512 // bp

    def _out_chunk(c):
        mxu = c % 2
        addr = ((c % 4) // 2) * 128
        p = pltpu.matmul_pop(addr, (512, 256), f32, mxu)
        o_buf[slot, pl.ds(c * otpc, otpc), :, :] = (
            p[:, :O].reshape(otpc, bp, O))

    for c in range(o_chunks):
        mxu = c % 2
        addr = ((c % 4) // 2) * 128
        pltpu.matmul_push_rhs(wo_ref[0:256, :], staging_register=0,
                              mxu_index=mxu)
        pltpu.matmul_push_rhs(wo_ref[256:512, :], staging_register=1,
                              mxu_index=mxu)
        lhs = rate_ref[pl.ds(c * 512, 512), :]
        pltpu.matmul_acc_lhs(addr, lhs[:, 0:256], mxu, load_staged_rhs=0)
        pltpu.matmul_acc_lhs(addr, lhs[:, 256:512], mxu, load_staged_rhs=1)
        if c > 1:
            _out_chunk(c - 2)
    _out_chunk(o_chunks - 2)
    _out_chunk(o_chunks - 1)

    for rb in range(b):
        o_tile_dma(slot, t, rb).start()

    @pl.when(t == nt - 1)
    def _epilogue():
        for rb in range(b):
            o_tile_dma(slot, 0, rb).wait()
        if nt > 1:
            for rb in range(b):
                o_tile_dma((nt - 2) % 2, 0, rb).wait()


def kernel(x, noise_tm, wi, si, wrec, wo, so, h0):
    B, S, I = x.shape
    H = wrec.shape[0]
    O = wo.shape[1]

    wi_a = (_ALPHA * (wi * si[:, None])).astype(jnp.float32)
    wrec_ta = (_ALPHA * jnp.transpose(wrec)).astype(jnp.bfloat16)
    wo_full = (wo * so[None, :]).astype(jnp.bfloat16)
    wo_p = jnp.pad(wo_full, ((0, 0), (0, 256 - O)))
    h0_2d = h0.reshape(1, H).astype(jnp.float32)

    Bp = _round_up(max(B, 8), 8)

    TS = min(64, S)
    S_pad = _round_up(S, TS)
    NT = S_pad // TS

    x_p = jnp.pad(x, ((0, 0), (0, S_pad - S), (0, 0)))
    n_p = jnp.pad(noise_tm, ((0, S_pad - S), (0, Bp - B), (0, 0)))

    _kernel_fn = functools.partial(_rnn_kernel, b=B, bp=Bp, ts=TS, nt=NT)

    grid_spec = pltpu.PrefetchScalarGridSpec(
        num_scalar_prefetch=0,
        grid=(NT,),
        in_specs=[
            pl.BlockSpec(memory_space=pltpu.MemorySpace.HBM),
            pl.BlockSpec((TS, Bp, H), lambda t: (t, 0, 0)),
            pl.BlockSpec((I, H), lambda t: (0, 0)),
            pl.BlockSpec((H, H), lambda t: (0, 0)),
            pl.BlockSpec((H, 256), lambda t: (0, 0)),
            pl.BlockSpec((1, H), lambda t: (0, 0)),
        ],
        out_specs=pl.BlockSpec(memory_space=pltpu.MemorySpace.HBM),
        scratch_shapes=[
            pltpu.VMEM((Bp, H), jnp.float32),
            pltpu.VMEM((Bp, H), jnp.bfloat16),
            pltpu.VMEM((TS * Bp, H), jnp.float32),
            pltpu.VMEM((TS * Bp, H), jnp.bfloat16),
            pltpu.VMEM((2, TS, Bp, I), jnp.float32),
            pltpu.VMEM((2, TS, Bp, O), jnp.float32),
            pltpu.SemaphoreType.DMA((2,)),
            pltpu.SemaphoreType.DMA((2,)),
        ],
    )

    out_p = pl.pallas_call(
        _kernel_fn,
        out_shape=jax.ShapeDtypeStruct((B, S_pad, O), jnp.float32),
        grid_spec=grid_spec,
        compiler_params=pltpu.CompilerParams(
            dimension_semantics=("arbitrary",),
            vmem_limit_bytes=56 * 2**20),
    )(x_p, n_p, wi_a, wrec_ta, wo_p, h0_2d)

    return out_p[:, :S]
